# baseline (speedup 1.0000x reference)
import functools

import jax
import jax.numpy as jnp
from jax import lax
from jax.experimental import pallas as pl
from jax.experimental.pallas import tpu as pltpu
from jax.experimental.pallas import tpu_sc as plsc

FEAT_DIM = 512
NUM_IDS = 100000
B = 4096
L = 16
CHUNK = 16
NBUF = 6


@functools.cache
def _build_sc():
    info = plsc.get_sparse_core_info()
    nw = info.num_cores * info.num_subcores
    b_per_w = B // nw
    n_chunks = b_per_w // CHUNK
    vecs_per_row = FEAT_DIM // L

    mesh = plsc.VectorSubcoreMesh(core_axis_name="c", subcore_axis_name="s")

    @functools.partial(
        pl.kernel,
        mesh=mesh,
        out_type=jax.ShapeDtypeStruct((B, FEAT_DIM), jnp.float32),
        scratch_types=(
            [pltpu.VMEM((b_per_w,), jnp.int32)]
            + [pltpu.VMEM((CHUNK, FEAT_DIM), jnp.float32) for _ in range(2 * NBUF)]
            + [pltpu.SemaphoreType.DMA for _ in range(3 * NBUF + 1)]
        ),
    )
    def k(table_hbm, idx_hbm, feat_hbm, out_hbm, idx_v, *bufs_and_sems):
        rows = bufs_and_sems[:NBUF]
        feats = bufs_and_sems[NBUF:2 * NBUF]
        gsems = bufs_and_sems[2 * NBUF:3 * NBUF]
        fsems = bufs_and_sems[3 * NBUF:4 * NBUF]
        osems = bufs_and_sems[4 * NBUF:5 * NBUF]
        isem = bufs_and_sems[5 * NBUF]

        wid = lax.axis_index("s") * info.num_cores + lax.axis_index("c")
        base = wid * b_per_w

        feat_cp = [None] * n_chunks
        for c in range(NBUF):
            feat_cp[c] = pltpu.async_copy(
                feat_hbm.at[pl.ds(base + c * CHUNK, CHUNK)], feats[c], fsems[c])
        idx_cp = pltpu.async_copy(idx_hbm.at[pl.ds(base, b_per_w)], idx_v, isem)
        idx_cp.wait()
        gather_cp = [None] * n_chunks
        for c in range(NBUF):
            gather_cp[c] = pltpu.async_copy(
                table_hbm.at[idx_v.at[pl.ds(c * CHUNK, CHUNK)]], rows[c], gsems[c])

        out_cp = [None] * n_chunks
        for c in range(n_chunks):
            b = c % NBUF
            gather_cp[c].wait()
            feat_cp[c].wait()
            rb, fb = rows[b], feats[b]

            def blend_row(i, carry):
                for v in range(vecs_per_row):
                    sl = pl.ds(v * L, L)
                    fb[i, sl] = 0.7 * fb[i, sl] + 0.3 * rb[i, sl]
                return carry

            lax.fori_loop(0, CHUNK, blend_row, 0)
            out_cp[c] = pltpu.async_copy(
                fb, out_hbm.at[pl.ds(base + c * CHUNK, CHUNK)], osems[b])
            if c + NBUF < n_chunks:
                gather_cp[c + NBUF] = pltpu.async_copy(
                    table_hbm.at[idx_v.at[pl.ds((c + NBUF) * CHUNK, CHUNK)]],
                    rows[b], gsems[b])
                out_cp[c].wait()
                feat_cp[c + NBUF] = pltpu.async_copy(
                    feat_hbm.at[pl.ds(base + (c + NBUF) * CHUNK, CHUNK)],
                    feats[b], fsems[b])
        for c in range(max(0, n_chunks - NBUF), n_chunks):
            out_cp[c].wait()

    return k


def kernel(feat, modality, pids, global_proto, local_proto):
    del local_proto
    table = global_proto.reshape(2 * NUM_IDS, FEAT_DIM)
    flat_idx = modality * NUM_IDS + pids
    return _build_sc()(table, flat_idx, feat)
